# baseline (speedup 1.0000x reference)
import functools

import jax
import jax.numpy as jnp
from jax import lax
from jax.experimental import pallas as pl
from jax.experimental.pallas import tpu as pltpu
from jax.experimental.pallas import tpu_sc as plsc

VOCAB = 100000
EMBED_DIM = 64
BATCH = 4096
FIELDS = 100
NW = 32
FB = 4
NFB = FIELDS // FB
BC = 512
NBC = BATCH // BC
NITER = NFB * NBC
NSLOT = 4

_mesh = plsc.VectorSubcoreMesh(core_axis_name="c", subcore_axis_name="s")


@functools.partial(
    pl.kernel,
    mesh=_mesh,
    out_type=jax.ShapeDtypeStruct((FIELDS, EMBED_DIM, BATCH), jnp.float32),
    scratch_types=[
        pltpu.VMEM((VOCAB,), jnp.int32),
        pltpu.VMEM((NSLOT, FB, BC), jnp.int32),
        pltpu.VMEM((NSLOT, FB, 2, BC), jnp.float32),
        [pltpu.SemaphoreType.DMA] * NSLOT,
        [pltpu.SemaphoreType.DMA] * NSLOT,
    ],
    compiler_params=pltpu.CompilerParams(
        use_tc_tiling_on_sc=True, needs_layout_passes=False
    ),
)
def _gather_kernel(pk_hbm, zt_hbm, out_hbm, row_v, zb_v, ob_v, zsems, osems):
    w = lax.axis_index("s") * 2 + lax.axis_index("c")

    def z_src(k):
        fb = k // NBC
        bc = lax.rem(k, NBC)
        return zt_hbm.at[pl.ds(fb * FB, FB), pl.ds(bc * BC, BC)]

    def z_start(k, slot):
        pltpu.async_copy(z_src(k), zb_v.at[slot], zsems[slot])

    def z_wait(slot):
        pltpu.make_async_copy(z_src(0), zb_v.at[slot], zsems[slot]).wait()

    def o_dst(k):
        fb = k // NBC
        bc = lax.rem(k, NBC)
        return out_hbm.at[
            pl.ds(fb * FB, FB), pl.ds(2 * w, 2), pl.ds(bc * BC, BC)
        ]

    def o_start(k, slot):
        pltpu.async_copy(ob_v.at[slot], o_dst(k), osems[slot])

    def o_wait(slot):
        pltpu.make_async_copy(ob_v.at[slot], o_dst(0), osems[slot]).wait()

    def compute(slot):
        for fi in range(FB):

            @plsc.parallel_loop(0, BC, step=16, unroll=32)
            def _(i):
                idxv = zb_v[slot, fi, pl.ds(i, 16)]
                g = plsc.load_gather(row_v, [idxv])
                lo, hi = plsc.unpack(
                    plsc.bitcast(g, jnp.bfloat16),
                    format=plsc.PackFormat.INTERLEAVED,
                )
                ob_v[slot, fi, 0, pl.ds(i, 16)] = lo
                ob_v[slot, fi, 1, pl.ds(i, 16)] = hi

    for s in range(NSLOT - 1):
        z_start(s, s)
    pltpu.sync_copy(pk_hbm.at[w], row_v)

    def body(m, _):
        for slot in range(NSLOT):
            k = NSLOT * m + slot
            nxt = k + NSLOT - 1

            @pl.when(nxt < NITER)
            def _():
                z_start(nxt, (slot + NSLOT - 1) % NSLOT)

            z_wait(slot)

            @pl.when(m > 0)
            def _():
                o_wait(slot)

            compute(slot)
            o_start(k, slot)
        return 0

    lax.fori_loop(0, NITER // NSLOT, body, 0)
    for s in range(NSLOT):
        o_wait(s)


def kernel(properties, z):
    bits = lax.bitcast_convert_type(properties, jnp.uint32)
    r = bits + jnp.uint32(0x8000)
    pk = lax.bitcast_convert_type(
        (r[:, 0::2] >> 16) | (r[:, 1::2] & jnp.uint32(0xFFFF0000)),
        jnp.int32,
    )
    out = _gather_kernel(pk.T, z.astype(jnp.int32).T)
    return out.transpose(2, 0, 1)


# speedup vs baseline: 2.0609x; 2.0609x over previous
import functools

import jax
import jax.numpy as jnp
from jax import lax
from jax.experimental import pallas as pl
from jax.experimental.pallas import tpu as pltpu
from jax.experimental.pallas import tpu_sc as plsc

VOCAB = 100000
EMBED_DIM = 64
BATCH = 4096
FIELDS = 100
NW = 32
FB = 4
NFB = FIELDS // FB
BC = 512
NBC = BATCH // BC
NITER = NFB * NBC
NSLOT = 4

_mesh = plsc.VectorSubcoreMesh(core_axis_name="c", subcore_axis_name="s")


@functools.partial(
    pl.kernel,
    mesh=_mesh,
    out_type=jax.ShapeDtypeStruct((FIELDS, EMBED_DIM, BATCH), jnp.float32),
    scratch_types=[
        pltpu.VMEM((VOCAB,), jnp.int32),
        pltpu.VMEM((NSLOT, FB, BC), jnp.int32),
        pltpu.VMEM((NSLOT, 2, FB, 1, BC), jnp.float32),
        [pltpu.SemaphoreType.DMA] * NSLOT,
        [pltpu.SemaphoreType.DMA] * NSLOT,
    ],
    compiler_params=pltpu.CompilerParams(
        use_tc_tiling_on_sc=True, needs_layout_passes=False
    ),
)
def _gather_kernel(pk_hbm, zt_hbm, out_hbm, row_v, zb_v, ob_v, zsems, osems):
    w = lax.axis_index("s") * 2 + lax.axis_index("c")

    def z_src(k):
        fb = k // NBC
        bc = lax.rem(k, NBC)
        return zt_hbm.at[pl.ds(fb * FB, FB), pl.ds(bc * BC, BC)]

    def z_start(k, slot):
        pltpu.async_copy(z_src(k), zb_v.at[slot], zsems[slot])

    def z_wait(slot):
        pltpu.make_async_copy(z_src(0), zb_v.at[slot], zsems[slot]).wait()

    def o_dst(k, r):
        fb = k // NBC
        bc = lax.rem(k, NBC)
        return out_hbm.at[
            pl.ds(fb * FB, FB), pl.ds(r, 1), pl.ds(bc * BC, BC)
        ]

    def o_start(k, slot):
        pltpu.async_copy(ob_v.at[slot, 0], o_dst(k, w), osems[slot])
        pltpu.async_copy(ob_v.at[slot, 1], o_dst(k, w + 32), osems[slot])

    def o_wait(slot):
        pltpu.make_async_copy(ob_v.at[slot, 0], o_dst(0, w), osems[slot]).wait()
        pltpu.make_async_copy(ob_v.at[slot, 1], o_dst(0, w), osems[slot]).wait()

    def compute(slot):
        for fi in range(FB):

            @plsc.parallel_loop(0, BC, step=16, unroll=32)
            def _(i):
                idxv = zb_v[slot, fi, pl.ds(i, 16)]
                g = plsc.load_gather(row_v, [idxv])
                lo, hi = plsc.unpack(
                    plsc.bitcast(g, jnp.bfloat16),
                    format=plsc.PackFormat.INTERLEAVED,
                )
                ob_v[slot, 0, fi, 0, pl.ds(i, 16)] = lo
                ob_v[slot, 1, fi, 0, pl.ds(i, 16)] = hi

    for s in range(NSLOT - 1):
        z_start(s, s)
    pltpu.sync_copy(pk_hbm.at[w], row_v)

    def body(m, _):
        for slot in range(NSLOT):
            k = NSLOT * m + slot
            nxt = k + NSLOT - 1

            @pl.when(nxt < NITER)
            def _():
                z_start(nxt, (slot + NSLOT - 1) % NSLOT)

            z_wait(slot)

            @pl.when(m > 0)
            def _():
                o_wait(slot)

            compute(slot)
            o_start(k, slot)
        return 0

    lax.fori_loop(0, NITER // NSLOT, body, 0)
    for s in range(NSLOT):
        o_wait(s)


def kernel(properties, z):
    bits = lax.bitcast_convert_type(properties, jnp.uint32)
    r = bits + jnp.uint32(0x8000)
    pk = lax.bitcast_convert_type(
        (r[:, :32] >> 16) | (r[:, 32:] & jnp.uint32(0xFFFF0000)),
        jnp.int32,
    )
    out = _gather_kernel(pk.T, z.astype(jnp.int32).T)
    return out.transpose(2, 0, 1)
